# Initial kernel scaffold; baseline (speedup 1.0000x reference)
import jax
import jax.numpy as jnp
from jax.experimental import pallas as pl


def kernel(x, edge_index, W1, b1, Wmu, bmu, Wlv, blv, Wc, bc, Wf, bf, Wl1, bl1, Wl2, bl2):
    raise NotImplementedError("write your pallas kernel here")


# speedup vs baseline: 6.6998x; 6.6998x over previous
import functools

import jax
import jax.numpy as jnp
from jax import lax
from jax.experimental import pallas as pl
from jax.experimental.pallas import tpu as pltpu
from jax.experimental.pallas import tpu_sc as plsc

N = 10000
IN = 128
HID = 128
LAT = 64
FEAT = 128
NC = 16
E = 160000

NP = 10240
EP = 163840
NCORE = 2
NSUB = 16
NW = NCORE * NSUB
EPT = EP // NW
CHUNK = 128
NCHUNK = EPT // CHUNK
RPT = NP // NSUB
DEGW = 128

_mesh = plsc.VectorSubcoreMesh(core_axis_name="c", subcore_axis_name="s")
f32 = jnp.float32


def _deg_body(dst_hbm, ones_hbm, zrows_hbm, out_hbm, idx_v, ones_v, acc_sh):
    c = lax.axis_index("c")
    s = lax.axis_index("s")
    pltpu.sync_copy(zrows_hbm, acc_sh.at[pl.ds(s * RPT, RPT)])
    pltpu.sync_copy(ones_hbm, ones_v)
    plsc.subcore_barrier()
    base = (c * NSUB + s) * EPT

    def body(i, carry):
        pltpu.sync_copy(dst_hbm.at[pl.ds(base + i * CHUNK, CHUNK)], idx_v)
        pltpu.sync_copy(ones_v, acc_sh.at[idx_v], add=True)
        return carry

    lax.fori_loop(0, NCHUNK, body, 0)
    plsc.subcore_barrier()
    pltpu.sync_copy(acc_sh.at[pl.ds(s * RPT, RPT)],
                    out_hbm.at[pl.ds(c * NP + s * RPT, RPT)])


_deg_call = functools.partial(
    pl.kernel,
    out_type=jax.ShapeDtypeStruct((NCORE * NP, DEGW), f32),
    mesh=_mesh,
    scratch_types=[
        pltpu.VMEM((CHUNK,), jnp.int32),
        pltpu.VMEM((CHUNK, DEGW), f32),
        pltpu.VMEM_SHARED((NP, DEGW), f32),
    ],
)(_deg_body)


def _agg_body(hs_hbm, src_hbm, dst_hbm, zrows_hbm, out_hbm,
              idxs_v, idxd_v, rows_v, acc_sh, sem):
    c = lax.axis_index("c")
    s = lax.axis_index("s")
    pltpu.sync_copy(zrows_hbm, acc_sh.at[pl.ds(s * RPT, RPT)])
    plsc.subcore_barrier()
    base = (c * NSUB + s) * EPT

    def body(i, carry):
        off = base + i * CHUNK
        pltpu.sync_copy(src_hbm.at[pl.ds(off, CHUNK)], idxs_v)
        pltpu.sync_copy(dst_hbm.at[pl.ds(off, CHUNK)], idxd_v)
        pltpu.async_copy(hs_hbm.at[idxs_v], rows_v, sem).wait()
        pltpu.sync_copy(rows_v, acc_sh.at[idxd_v], add=True)
        return carry

    lax.fori_loop(0, NCHUNK, body, 0)
    plsc.subcore_barrier()
    pltpu.sync_copy(acc_sh.at[pl.ds(s * RPT, RPT)],
                    out_hbm.at[pl.ds(c * NP + s * RPT, RPT)])


_agg_call = functools.partial(
    pl.kernel,
    out_type=jax.ShapeDtypeStruct((NCORE * NP, HID), f32),
    mesh=_mesh,
    scratch_types=[
        pltpu.VMEM((CHUNK,), jnp.int32),
        pltpu.VMEM((CHUNK,), jnp.int32),
        pltpu.VMEM((CHUNK, HID), f32),
        pltpu.VMEM_SHARED((NP, HID), f32),
        pltpu.SemaphoreType.DMA,
    ],
)(_agg_body)


BLK = 1024

_tc_params = pltpu.CompilerParams(dimension_semantics=("parallel",))


def _tc1_body(x_ref, w1_ref, wc_ref, bc_ref, h0_ref, cp_ref):
    xb = x_ref[...]
    h0_ref[...] = jnp.dot(xb, w1_ref[...], preferred_element_type=f32)
    cp_ref[...] = jnp.maximum(
        jnp.dot(xb, wc_ref[...], preferred_element_type=f32) + bc_ref[...], 0.0)


def _tc1_call(xp, W1, Wc, bc2):
    return pl.pallas_call(
        _tc1_body,
        grid=(NP // BLK,),
        in_specs=[
            pl.BlockSpec((BLK, IN), lambda i: (i, 0)),
            pl.BlockSpec((IN, HID), lambda i: (0, 0)),
            pl.BlockSpec((IN, LAT), lambda i: (0, 0)),
            pl.BlockSpec((1, LAT), lambda i: (0, 0)),
        ],
        out_specs=[
            pl.BlockSpec((BLK, HID), lambda i: (i, 0)),
            pl.BlockSpec((BLK, LAT), lambda i: (i, 0)),
        ],
        out_shape=[
            jax.ShapeDtypeStruct((NP, HID), f32),
            jax.ShapeDtypeStruct((NP, LAT), f32),
        ],
        compiler_params=_tc_params,
    )(xp, W1, Wc, bc2)


def _tc2_body(d0_ref, d1_ref, h0_ref, dinv_ref, hs0_ref):
    deg = jnp.maximum(d0_ref[:, :1] + d1_ref[:, :1] + 1.0, 1.0)
    dinv = 1.0 / jnp.sqrt(deg)
    dinv_ref[...] = dinv
    hs0_ref[...] = h0_ref[...] * dinv


def _tc2_call(d0, d1, h0):
    return pl.pallas_call(
        _tc2_body,
        grid=(NP // BLK,),
        in_specs=[
            pl.BlockSpec((BLK, DEGW), lambda i: (i, 0)),
            pl.BlockSpec((BLK, DEGW), lambda i: (i, 0)),
            pl.BlockSpec((BLK, HID), lambda i: (i, 0)),
        ],
        out_specs=[
            pl.BlockSpec((BLK, 1), lambda i: (i, 0)),
            pl.BlockSpec((BLK, HID), lambda i: (i, 0)),
        ],
        out_shape=[
            jax.ShapeDtypeStruct((NP, 1), f32),
            jax.ShapeDtypeStruct((NP, HID), f32),
        ],
        compiler_params=_tc_params,
    )(d0, d1, h0)


def _tc3_body(a0_ref, a1_ref, hs0_ref, dinv_ref, b1_ref, hs1_ref):
    dinv = dinv_ref[...]
    g1 = dinv * (a0_ref[...] + a1_ref[...] + hs0_ref[...]) + b1_ref[...]
    hs1_ref[...] = jnp.maximum(g1, 0.0) * dinv


def _tc3_call(a0, a1, hs0, dinv, b12):
    return pl.pallas_call(
        _tc3_body,
        grid=(NP // BLK,),
        in_specs=[
            pl.BlockSpec((BLK, HID), lambda i: (i, 0)),
            pl.BlockSpec((BLK, HID), lambda i: (i, 0)),
            pl.BlockSpec((BLK, HID), lambda i: (i, 0)),
            pl.BlockSpec((BLK, 1), lambda i: (i, 0)),
            pl.BlockSpec((1, HID), lambda i: (0, 0)),
        ],
        out_specs=pl.BlockSpec((BLK, HID), lambda i: (i, 0)),
        out_shape=jax.ShapeDtypeStruct((NP, HID), f32),
        compiler_params=_tc_params,
    )(a0, a1, hs0, dinv, b12)


def _tc4_body(q0_ref, q1_ref, hs1_ref, dinv_ref, cp_ref,
              wmu_ref, bmu_ref, wlv_ref, blv_ref, wf_ref, bf_ref,
              wl1_ref, bl1_ref, wl2_ref, bl2_ref,
              mu_ref, lv_ref, hc_ref, xp_ref, y_ref):
    g2 = dinv_ref[...] * (q0_ref[...] + q1_ref[...] + hs1_ref[...])
    mu = jnp.dot(g2, wmu_ref[...], preferred_element_type=f32) + bmu_ref[...]
    mu_ref[...] = mu
    lv_ref[...] = jnp.dot(g2, wlv_ref[...], preferred_element_type=f32) + blv_ref[...]
    hc_ref[...] = mu + cp_ref[...]
    xp_ref[...] = jnp.dot(mu, wf_ref[...], preferred_element_type=f32) + bf_ref[...]
    t = jnp.maximum(
        jnp.dot(mu, wl1_ref[...], preferred_element_type=f32) + bl1_ref[...], 0.0)
    y_ref[...] = jnp.dot(t, wl2_ref[...], preferred_element_type=f32) + bl2_ref[...]


def _tc4_call(q0, q1, hs1, dinv, cp, Wmu, bmu2, Wlv, blv2, Wf, bf2,
              Wl1, bl12, Wl2, bl22):
    row = lambda w: pl.BlockSpec((BLK, w), lambda i: (i, 0))
    full = lambda a, b: pl.BlockSpec((a, b), lambda i: (0, 0))
    return pl.pallas_call(
        _tc4_body,
        grid=(NP // BLK,),
        in_specs=[
            row(HID), row(HID), row(HID), pl.BlockSpec((BLK, 1), lambda i: (i, 0)),
            row(LAT),
            full(HID, LAT), full(1, LAT), full(HID, LAT), full(1, LAT),
            full(LAT, FEAT), full(1, FEAT), full(LAT, HID), full(1, HID),
            full(HID, NC), full(1, NC),
        ],
        out_specs=[row(LAT), row(LAT), row(LAT), row(FEAT), row(NC)],
        out_shape=[
            jax.ShapeDtypeStruct((NP, LAT), f32),
            jax.ShapeDtypeStruct((NP, LAT), f32),
            jax.ShapeDtypeStruct((NP, LAT), f32),
            jax.ShapeDtypeStruct((NP, FEAT), f32),
            jax.ShapeDtypeStruct((NP, NC), f32),
        ],
        compiler_params=_tc_params,
    )(q0, q1, hs1, dinv, cp, Wmu, bmu2, Wlv, blv2, Wf, bf2, Wl1, bl12, Wl2, bl22)


ABLK = 200


def _tc5_body(a_ref, b_ref, o_ref):
    prod = lax.dot_general(a_ref[...], b_ref[...],
                           (((1,), (1,)), ((), ())),
                           preferred_element_type=f32)
    o_ref[...] = jax.nn.sigmoid(prod)


def _tc5_call(hc):
    return pl.pallas_call(
        _tc5_body,
        grid=(N // ABLK,),
        in_specs=[
            pl.BlockSpec((ABLK, LAT), lambda i: (i, 0)),
            pl.BlockSpec((N, LAT), lambda i: (0, 0)),
        ],
        out_specs=pl.BlockSpec((ABLK, N), lambda i: (i, 0)),
        out_shape=jax.ShapeDtypeStruct((N, N), f32),
        compiler_params=_tc_params,
    )(hc, hc)


def kernel(x, edge_index, W1, b1, Wmu, bmu, Wlv, blv, Wc, bc, Wf, bf,
           Wl1, bl1, Wl2, bl2):
    xp = jnp.pad(x, ((0, NP - N), (0, 0)))
    pad = jnp.full((EP - E,), N, jnp.int32)
    srcp = jnp.concatenate([edge_index[0], pad])
    dstp = jnp.concatenate([edge_index[1], pad])
    zrows = jnp.zeros((RPT, HID), f32)
    zdeg = jnp.zeros((RPT, DEGW), f32)
    ones_deg = jnp.ones((CHUNK, DEGW), f32)

    degp = _deg_call(dstp, ones_deg, zdeg)
    h0, cpart = _tc1_call(xp, W1, Wc, bc.reshape(1, LAT))
    dinv, hs0 = _tc2_call(degp[:NP], degp[NP:], h0)
    accp = _agg_call(hs0, srcp, dstp, zrows)
    hs1 = _tc3_call(accp[:NP], accp[NP:], hs0, dinv, b1.reshape(1, HID))
    qp = _agg_call(hs1, srcp, dstp, zrows)
    mu, lv, hc, xpred, y = _tc4_call(
        qp[:NP], qp[NP:], hs1, dinv, cpart,
        Wmu, bmu.reshape(1, LAT), Wlv, blv.reshape(1, LAT),
        Wf, bf.reshape(1, FEAT), Wl1, bl1.reshape(1, HID),
        Wl2, bl2.reshape(1, NC))
    A = _tc5_call(hc[:N])
    mu = mu[:N]
    return (mu, lv[:N], mu, A, xpred[:N], y[:N])


# speedup vs baseline: 7.7901x; 1.1628x over previous
import functools

import jax
import jax.numpy as jnp
from jax import lax
from jax.experimental import pallas as pl
from jax.experimental.pallas import tpu as pltpu
from jax.experimental.pallas import tpu_sc as plsc

N = 10000
IN = 128
HID = 128
LAT = 64
FEAT = 128
NC = 16
E = 160000

NP = 10240
EP = 163840
NCORE = 2
NSUB = 16
NW = NCORE * NSUB
EPT = EP // NW
CHUNK = 128
NCHUNK = EPT // CHUNK
RPT = NP // NSUB
DEGW = 128

_mesh = plsc.VectorSubcoreMesh(core_axis_name="c", subcore_axis_name="s")
f32 = jnp.float32


def _deg_body(dst_hbm, ones_hbm, zrows_hbm, out_hbm, idx_v, ones_v, acc_sh):
    c = lax.axis_index("c")
    s = lax.axis_index("s")
    wid = c * NSUB + s
    pltpu.sync_copy(zrows_hbm, acc_sh.at[pl.ds(s * RPT, RPT)])
    pltpu.sync_copy(ones_hbm, ones_v)
    pltpu.sync_copy(dst_hbm.at[wid], idx_v)
    plsc.subcore_barrier()

    def body(i, carry):
        pltpu.sync_copy(ones_v, acc_sh.at[idx_v.at[i]], add=True)
        return carry

    lax.fori_loop(0, NCHUNK, body, 0)
    plsc.subcore_barrier()
    pltpu.sync_copy(acc_sh.at[pl.ds(s * RPT, RPT)],
                    out_hbm.at[pl.ds(c * NP + s * RPT, RPT)])


_deg_call = functools.partial(
    pl.kernel,
    out_type=jax.ShapeDtypeStruct((NCORE * NP, DEGW), f32),
    mesh=_mesh,
    scratch_types=[
        pltpu.VMEM((NCHUNK, CHUNK), jnp.int32),
        pltpu.VMEM((CHUNK, DEGW), f32),
        pltpu.VMEM_SHARED((NP, DEGW), f32),
    ],
)(_deg_body)


NBUF = 2
NGROUP = NCHUNK // NBUF


def _agg_body(hs_hbm, src_hbm, dst_hbm, zrows_hbm, out_hbm,
              idxs_v, idxd_v, rows0, rows1, acc_sh, sem0, sem1):
    rows = (rows0, rows1)
    sems = (sem0, sem1)
    c = lax.axis_index("c")
    s = lax.axis_index("s")
    wid = c * NSUB + s
    pltpu.sync_copy(zrows_hbm, acc_sh.at[pl.ds(s * RPT, RPT)])
    pltpu.sync_copy(src_hbm.at[wid], idxs_v)
    pltpu.sync_copy(dst_hbm.at[wid], idxd_v)
    plsc.subcore_barrier()

    def fire(chunk, b):
        pltpu.async_copy(hs_hbm.at[idxs_v.at[chunk]], rows[b], sems[b])

    for b in range(NBUF):
        fire(b, b)

    def gbody(g, carry):
        for b in range(NBUF):
            chunk = g * NBUF + b
            pltpu.make_async_copy(hs_hbm.at[idxs_v.at[chunk]],
                                  rows[b], sems[b]).wait()
            pltpu.sync_copy(rows[b], acc_sh.at[idxd_v.at[chunk]], add=True)

            @pl.when(g + 1 < NGROUP)
            def _():
                fire(chunk + NBUF, b)
        return carry

    lax.fori_loop(0, NGROUP, gbody, 0)
    plsc.subcore_barrier()
    pltpu.sync_copy(acc_sh.at[pl.ds(s * RPT, RPT)],
                    out_hbm.at[pl.ds(c * NP + s * RPT, RPT)])


_agg_call = functools.partial(
    pl.kernel,
    out_type=jax.ShapeDtypeStruct((NCORE * NP, HID), f32),
    mesh=_mesh,
    scratch_types=[
        pltpu.VMEM((NCHUNK, CHUNK), jnp.int32),
        pltpu.VMEM((NCHUNK, CHUNK), jnp.int32),
        pltpu.VMEM((CHUNK, HID), f32),
        pltpu.VMEM((CHUNK, HID), f32),
        pltpu.VMEM_SHARED((NP, HID), f32),
        pltpu.SemaphoreType.DMA,
        pltpu.SemaphoreType.DMA,
    ],
)(_agg_body)


BLK = 1024

_tc_params = pltpu.CompilerParams(dimension_semantics=("parallel",))


def _tc1_body(x_ref, w1_ref, wc_ref, bc_ref, h0_ref, cp_ref):
    xb = x_ref[...]
    h0_ref[...] = jnp.dot(xb, w1_ref[...], preferred_element_type=f32)
    cp_ref[...] = jnp.maximum(
        jnp.dot(xb, wc_ref[...], preferred_element_type=f32) + bc_ref[...], 0.0)


def _tc1_call(xp, W1, Wc, bc2):
    return pl.pallas_call(
        _tc1_body,
        grid=(NP // BLK,),
        in_specs=[
            pl.BlockSpec((BLK, IN), lambda i: (i, 0)),
            pl.BlockSpec((IN, HID), lambda i: (0, 0)),
            pl.BlockSpec((IN, LAT), lambda i: (0, 0)),
            pl.BlockSpec((1, LAT), lambda i: (0, 0)),
        ],
        out_specs=[
            pl.BlockSpec((BLK, HID), lambda i: (i, 0)),
            pl.BlockSpec((BLK, LAT), lambda i: (i, 0)),
        ],
        out_shape=[
            jax.ShapeDtypeStruct((NP, HID), f32),
            jax.ShapeDtypeStruct((NP, LAT), f32),
        ],
        compiler_params=_tc_params,
    )(xp, W1, Wc, bc2)


def _tc2_body(d0_ref, d1_ref, h0_ref, dinv_ref, hs0_ref):
    deg = jnp.maximum(d0_ref[:, :1] + d1_ref[:, :1] + 1.0, 1.0)
    dinv = 1.0 / jnp.sqrt(deg)
    dinv_ref[...] = dinv
    hs0_ref[...] = h0_ref[...] * dinv


def _tc2_call(d0, d1, h0):
    return pl.pallas_call(
        _tc2_body,
        grid=(NP // BLK,),
        in_specs=[
            pl.BlockSpec((BLK, DEGW), lambda i: (i, 0)),
            pl.BlockSpec((BLK, DEGW), lambda i: (i, 0)),
            pl.BlockSpec((BLK, HID), lambda i: (i, 0)),
        ],
        out_specs=[
            pl.BlockSpec((BLK, 1), lambda i: (i, 0)),
            pl.BlockSpec((BLK, HID), lambda i: (i, 0)),
        ],
        out_shape=[
            jax.ShapeDtypeStruct((NP, 1), f32),
            jax.ShapeDtypeStruct((NP, HID), f32),
        ],
        compiler_params=_tc_params,
    )(d0, d1, h0)


def _tc3_body(a0_ref, a1_ref, hs0_ref, dinv_ref, b1_ref, hs1_ref):
    dinv = dinv_ref[...]
    g1 = dinv * (a0_ref[...] + a1_ref[...] + hs0_ref[...]) + b1_ref[...]
    hs1_ref[...] = jnp.maximum(g1, 0.0) * dinv


def _tc3_call(a0, a1, hs0, dinv, b12):
    return pl.pallas_call(
        _tc3_body,
        grid=(NP // BLK,),
        in_specs=[
            pl.BlockSpec((BLK, HID), lambda i: (i, 0)),
            pl.BlockSpec((BLK, HID), lambda i: (i, 0)),
            pl.BlockSpec((BLK, HID), lambda i: (i, 0)),
            pl.BlockSpec((BLK, 1), lambda i: (i, 0)),
            pl.BlockSpec((1, HID), lambda i: (0, 0)),
        ],
        out_specs=pl.BlockSpec((BLK, HID), lambda i: (i, 0)),
        out_shape=jax.ShapeDtypeStruct((NP, HID), f32),
        compiler_params=_tc_params,
    )(a0, a1, hs0, dinv, b12)


def _tc4_body(q0_ref, q1_ref, hs1_ref, dinv_ref, cp_ref,
              wmu_ref, bmu_ref, wlv_ref, blv_ref, wf_ref, bf_ref,
              wl1_ref, bl1_ref, wl2_ref, bl2_ref,
              mu_ref, lv_ref, hc_ref, xp_ref, y_ref):
    g2 = dinv_ref[...] * (q0_ref[...] + q1_ref[...] + hs1_ref[...])
    mu = jnp.dot(g2, wmu_ref[...], preferred_element_type=f32) + bmu_ref[...]
    mu_ref[...] = mu
    lv_ref[...] = jnp.dot(g2, wlv_ref[...], preferred_element_type=f32) + blv_ref[...]
    hc_ref[...] = mu + cp_ref[...]
    xp_ref[...] = jnp.dot(mu, wf_ref[...], preferred_element_type=f32) + bf_ref[...]
    t = jnp.maximum(
        jnp.dot(mu, wl1_ref[...], preferred_element_type=f32) + bl1_ref[...], 0.0)
    y_ref[...] = jnp.dot(t, wl2_ref[...], preferred_element_type=f32) + bl2_ref[...]


def _tc4_call(q0, q1, hs1, dinv, cp, Wmu, bmu2, Wlv, blv2, Wf, bf2,
              Wl1, bl12, Wl2, bl22):
    row = lambda w: pl.BlockSpec((BLK, w), lambda i: (i, 0))
    full = lambda a, b: pl.BlockSpec((a, b), lambda i: (0, 0))
    return pl.pallas_call(
        _tc4_body,
        grid=(NP // BLK,),
        in_specs=[
            row(HID), row(HID), row(HID), pl.BlockSpec((BLK, 1), lambda i: (i, 0)),
            row(LAT),
            full(HID, LAT), full(1, LAT), full(HID, LAT), full(1, LAT),
            full(LAT, FEAT), full(1, FEAT), full(LAT, HID), full(1, HID),
            full(HID, NC), full(1, NC),
        ],
        out_specs=[row(LAT), row(LAT), row(LAT), row(FEAT), row(NC)],
        out_shape=[
            jax.ShapeDtypeStruct((NP, LAT), f32),
            jax.ShapeDtypeStruct((NP, LAT), f32),
            jax.ShapeDtypeStruct((NP, LAT), f32),
            jax.ShapeDtypeStruct((NP, FEAT), f32),
            jax.ShapeDtypeStruct((NP, NC), f32),
        ],
        compiler_params=_tc_params,
    )(q0, q1, hs1, dinv, cp, Wmu, bmu2, Wlv, blv2, Wf, bf2, Wl1, bl12, Wl2, bl22)


ABLK = 200


def _tc5_body(a_ref, b_ref, o_ref):
    prod = lax.dot_general(a_ref[...], b_ref[...],
                           (((1,), (1,)), ((), ())),
                           preferred_element_type=f32)
    o_ref[...] = jax.nn.sigmoid(prod)


def _tc5_call(hc):
    return pl.pallas_call(
        _tc5_body,
        grid=(N // ABLK,),
        in_specs=[
            pl.BlockSpec((ABLK, LAT), lambda i: (i, 0)),
            pl.BlockSpec((N, LAT), lambda i: (0, 0)),
        ],
        out_specs=pl.BlockSpec((ABLK, N), lambda i: (i, 0)),
        out_shape=jax.ShapeDtypeStruct((N, N), f32),
        compiler_params=_tc_params,
    )(hc, hc)


def kernel(x, edge_index, W1, b1, Wmu, bmu, Wlv, blv, Wc, bc, Wf, bf,
           Wl1, bl1, Wl2, bl2):
    xp = jnp.pad(x, ((0, NP - N), (0, 0)))
    pad = jnp.full((EP - E,), N, jnp.int32)
    srcp = jnp.concatenate([edge_index[0], pad]).reshape(NW, NCHUNK, CHUNK)
    dstp = jnp.concatenate([edge_index[1], pad]).reshape(NW, NCHUNK, CHUNK)
    zrows = jnp.zeros((RPT, HID), f32)
    zdeg = jnp.zeros((RPT, DEGW), f32)
    ones_deg = jnp.ones((CHUNK, DEGW), f32)

    degp = _deg_call(dstp, ones_deg, zdeg)
    h0, cpart = _tc1_call(xp, W1, Wc, bc.reshape(1, LAT))
    dinv, hs0 = _tc2_call(degp[:NP], degp[NP:], h0)
    accp = _agg_call(hs0, srcp, dstp, zrows)
    hs1 = _tc3_call(accp[:NP], accp[NP:], hs0, dinv, b1.reshape(1, HID))
    qp = _agg_call(hs1, srcp, dstp, zrows)
    mu, lv, hc, xpred, y = _tc4_call(
        qp[:NP], qp[NP:], hs1, dinv, cpart,
        Wmu, bmu.reshape(1, LAT), Wlv, blv.reshape(1, LAT),
        Wf, bf.reshape(1, FEAT), Wl1, bl1.reshape(1, HID),
        Wl2, bl2.reshape(1, NC))
    A = _tc5_call(hc[:N])
    mu = mu[:N]
    return (mu, lv[:N], mu, A, xpred[:N], y[:N])


# speedup vs baseline: 8.3010x; 1.0656x over previous
import functools

import jax
import jax.numpy as jnp
from jax import lax
from jax.experimental import pallas as pl
from jax.experimental.pallas import tpu as pltpu
from jax.experimental.pallas import tpu_sc as plsc

N = 10000
IN = 128
HID = 128
LAT = 64
FEAT = 128
NC = 16
E = 160000

NP = 10240
EP = 163840
NCORE = 2
NSUB = 16
NW = NCORE * NSUB
CHUNK = 128
TCHUNK = EP // CHUNK
RPT = NP // NSUB
DEGW = 128
NBUF = 2
NC0 = 56
NC1 = 24
NDEG = TCHUNK // NW

_mesh = plsc.VectorSubcoreMesh(core_axis_name="c", subcore_axis_name="s")
f32 = jnp.float32


def _deg_body(dst_hbm, ones_hbm, zrows_hbm, out_hbm, idx_v, ones_v, acc_sh):
    c = lax.axis_index("c")
    s = lax.axis_index("s")
    wid = c * NSUB + s
    rsl = pl.ds(s * RPT, RPT)
    pltpu.sync_copy(zrows_hbm, acc_sh.at[rsl])
    pltpu.sync_copy(ones_hbm, ones_v)
    pltpu.sync_copy(dst_hbm.at[pl.ds(wid * NDEG, NDEG)], idx_v)
    plsc.subcore_barrier()

    def body(i, carry):
        pltpu.sync_copy(ones_v, acc_sh.at[idx_v.at[i]], add=True)
        return carry

    lax.fori_loop(0, NDEG, body, 0)
    plsc.subcore_barrier()
    pltpu.sync_copy(acc_sh.at[rsl], out_hbm.at[pl.ds(c * NP + s * RPT, RPT)])


_deg_call = functools.partial(
    pl.kernel,
    out_type=jax.ShapeDtypeStruct((NCORE * NP, DEGW), f32),
    mesh=_mesh,
    scratch_types=[
        pltpu.VMEM((NDEG, CHUNK), jnp.int32),
        pltpu.VMEM((CHUNK, DEGW), f32),
        pltpu.VMEM_SHARED((NP, DEGW), f32),
    ],
)(_deg_body)


def _agg_body(hs_hbm, src_hbm, dst_hbm, zrows_hbm, out_hbm,
              idxs_v, idxd_v, rows0, rows1, acc_sh, sem0, sem1):
    rows = (rows0, rows1)
    sems = (sem0, sem1)
    c = lax.axis_index("c")
    s = lax.axis_index("s")
    rsl = pl.ds(s * RPT, RPT)
    pltpu.sync_copy(zrows_hbm, acc_sh.at[rsl])

    def run(row0, nchunks):
        pltpu.sync_copy(src_hbm.at[pl.ds(row0, nchunks)],
                        idxs_v.at[pl.ds(0, nchunks)])
        pltpu.sync_copy(dst_hbm.at[pl.ds(row0, nchunks)],
                        idxd_v.at[pl.ds(0, nchunks)])
        plsc.subcore_barrier()

        def fire(chunk, b):
            pltpu.async_copy(hs_hbm.at[idxs_v.at[chunk]], rows[b], sems[b])

        for b in range(NBUF):
            fire(b, b)
        ngroup = nchunks // NBUF

        def gbody(g, carry):
            for b in range(NBUF):
                chunk = g * NBUF + b
                pltpu.make_async_copy(hs_hbm.at[idxs_v.at[chunk]],
                                      rows[b], sems[b]).wait()
                pltpu.sync_copy(rows[b], acc_sh.at[idxd_v.at[chunk]], add=True)

                @pl.when(g + 1 < ngroup)
                def _():
                    fire(chunk + NBUF, b)
            return carry

        lax.fori_loop(0, ngroup, gbody, 0)

    @pl.when(c == 0)
    def _():
        run(s * NC0, NC0)

    @pl.when(c == 1)
    def _():
        run(NSUB * NC0 + s * NC1, NC1)

    plsc.subcore_barrier()
    pltpu.sync_copy(acc_sh.at[rsl], out_hbm.at[pl.ds(c * NP + s * RPT, RPT)])


_agg_call = functools.partial(
    pl.kernel,
    out_type=jax.ShapeDtypeStruct((NCORE * NP, HID), f32),
    mesh=_mesh,
    scratch_types=[
        pltpu.VMEM((NC0, CHUNK), jnp.int32),
        pltpu.VMEM((NC0, CHUNK), jnp.int32),
        pltpu.VMEM((CHUNK, HID), f32),
        pltpu.VMEM((CHUNK, HID), f32),
        pltpu.VMEM_SHARED((NP, HID), f32),
        pltpu.SemaphoreType.DMA,
        pltpu.SemaphoreType.DMA,
    ],
)(_agg_body)


BLK = 1024

_tc_params = pltpu.CompilerParams(dimension_semantics=("parallel",))


def _tc1_body(x_ref, w1_ref, wc_ref, bc_ref, h0_ref, cp_ref):
    xb = x_ref[...]
    h0_ref[...] = jnp.dot(xb, w1_ref[...], preferred_element_type=f32)
    cp_ref[...] = jnp.maximum(
        jnp.dot(xb, wc_ref[...], preferred_element_type=f32) + bc_ref[...], 0.0)


def _tc1_call(xp, W1, Wc, bc2):
    return pl.pallas_call(
        _tc1_body,
        grid=(NP // BLK,),
        in_specs=[
            pl.BlockSpec((BLK, IN), lambda i: (i, 0)),
            pl.BlockSpec((IN, HID), lambda i: (0, 0)),
            pl.BlockSpec((IN, LAT), lambda i: (0, 0)),
            pl.BlockSpec((1, LAT), lambda i: (0, 0)),
        ],
        out_specs=[
            pl.BlockSpec((BLK, HID), lambda i: (i, 0)),
            pl.BlockSpec((BLK, LAT), lambda i: (i, 0)),
        ],
        out_shape=[
            jax.ShapeDtypeStruct((NP, HID), f32),
            jax.ShapeDtypeStruct((NP, LAT), f32),
        ],
        compiler_params=_tc_params,
    )(xp, W1, Wc, bc2)


def _tc2_body(d0_ref, d1_ref, h0_ref, dinv_ref, hs0_ref):
    deg = jnp.maximum(d0_ref[:, :1] + d1_ref[:, :1] + 1.0, 1.0)
    dinv = 1.0 / jnp.sqrt(deg)
    dinv_ref[...] = dinv
    hs0_ref[...] = h0_ref[...] * dinv


def _tc2_call(d0, d1, h0):
    return pl.pallas_call(
        _tc2_body,
        grid=(NP // BLK,),
        in_specs=[
            pl.BlockSpec((BLK, DEGW), lambda i: (i, 0)),
            pl.BlockSpec((BLK, DEGW), lambda i: (i, 0)),
            pl.BlockSpec((BLK, HID), lambda i: (i, 0)),
        ],
        out_specs=[
            pl.BlockSpec((BLK, 1), lambda i: (i, 0)),
            pl.BlockSpec((BLK, HID), lambda i: (i, 0)),
        ],
        out_shape=[
            jax.ShapeDtypeStruct((NP, 1), f32),
            jax.ShapeDtypeStruct((NP, HID), f32),
        ],
        compiler_params=_tc_params,
    )(d0, d1, h0)


def _tc3_body(a0_ref, a1_ref, hs0_ref, dinv_ref, b1_ref, hs1_ref):
    dinv = dinv_ref[...]
    g1 = dinv * (a0_ref[...] + a1_ref[...] + hs0_ref[...]) + b1_ref[...]
    hs1_ref[...] = jnp.maximum(g1, 0.0) * dinv


def _tc3_call(a0, a1, hs0, dinv, b12):
    return pl.pallas_call(
        _tc3_body,
        grid=(NP // BLK,),
        in_specs=[
            pl.BlockSpec((BLK, HID), lambda i: (i, 0)),
            pl.BlockSpec((BLK, HID), lambda i: (i, 0)),
            pl.BlockSpec((BLK, HID), lambda i: (i, 0)),
            pl.BlockSpec((BLK, 1), lambda i: (i, 0)),
            pl.BlockSpec((1, HID), lambda i: (0, 0)),
        ],
        out_specs=pl.BlockSpec((BLK, HID), lambda i: (i, 0)),
        out_shape=jax.ShapeDtypeStruct((NP, HID), f32),
        compiler_params=_tc_params,
    )(a0, a1, hs0, dinv, b12)


def _tc4_body(q0_ref, q1_ref, hs1_ref, dinv_ref, cp_ref,
              wmu_ref, bmu_ref, wlv_ref, blv_ref, wf_ref, bf_ref,
              wl1_ref, bl1_ref, wl2_ref, bl2_ref,
              mu_ref, lv_ref, hc_ref, xp_ref, y_ref):
    g2 = dinv_ref[...] * (q0_ref[...] + q1_ref[...] + hs1_ref[...])
    mu = jnp.dot(g2, wmu_ref[...], preferred_element_type=f32) + bmu_ref[...]
    mu_ref[...] = mu
    lv_ref[...] = jnp.dot(g2, wlv_ref[...], preferred_element_type=f32) + blv_ref[...]
    hc_ref[...] = mu + cp_ref[...]
    xp_ref[...] = jnp.dot(mu, wf_ref[...], preferred_element_type=f32) + bf_ref[...]
    t = jnp.maximum(
        jnp.dot(mu, wl1_ref[...], preferred_element_type=f32) + bl1_ref[...], 0.0)
    y_ref[...] = jnp.dot(t, wl2_ref[...], preferred_element_type=f32) + bl2_ref[...]


def _tc4_call(q0, q1, hs1, dinv, cp, Wmu, bmu2, Wlv, blv2, Wf, bf2,
              Wl1, bl12, Wl2, bl22):
    row = lambda w: pl.BlockSpec((BLK, w), lambda i: (i, 0))
    full = lambda a, b: pl.BlockSpec((a, b), lambda i: (0, 0))
    return pl.pallas_call(
        _tc4_body,
        grid=(NP // BLK,),
        in_specs=[
            row(HID), row(HID), row(HID), pl.BlockSpec((BLK, 1), lambda i: (i, 0)),
            row(LAT),
            full(HID, LAT), full(1, LAT), full(HID, LAT), full(1, LAT),
            full(LAT, FEAT), full(1, FEAT), full(LAT, HID), full(1, HID),
            full(HID, NC), full(1, NC),
        ],
        out_specs=[row(LAT), row(LAT), row(LAT), row(FEAT), row(NC)],
        out_shape=[
            jax.ShapeDtypeStruct((NP, LAT), f32),
            jax.ShapeDtypeStruct((NP, LAT), f32),
            jax.ShapeDtypeStruct((NP, LAT), f32),
            jax.ShapeDtypeStruct((NP, FEAT), f32),
            jax.ShapeDtypeStruct((NP, NC), f32),
        ],
        compiler_params=_tc_params,
    )(q0, q1, hs1, dinv, cp, Wmu, bmu2, Wlv, blv2, Wf, bf2, Wl1, bl12, Wl2, bl22)


ABLK = 200


def _tc5_body(a_ref, b_ref, o_ref):
    prod = lax.dot_general(a_ref[...], b_ref[...],
                           (((1,), (1,)), ((), ())),
                           preferred_element_type=f32)
    o_ref[...] = jax.nn.sigmoid(prod)


def _tc5_call(hc):
    return pl.pallas_call(
        _tc5_body,
        grid=(N // ABLK,),
        in_specs=[
            pl.BlockSpec((ABLK, LAT), lambda i: (i, 0)),
            pl.BlockSpec((N, LAT), lambda i: (0, 0)),
        ],
        out_specs=pl.BlockSpec((ABLK, N), lambda i: (i, 0)),
        out_shape=jax.ShapeDtypeStruct((N, N), f32),
        compiler_params=_tc_params,
    )(hc, hc)


def kernel(x, edge_index, W1, b1, Wmu, bmu, Wlv, blv, Wc, bc, Wf, bf,
           Wl1, bl1, Wl2, bl2):
    xp = jnp.pad(x, ((0, NP - N), (0, 0)))
    pad = jnp.full((EP - E,), N, jnp.int32)
    srcp = jnp.concatenate([edge_index[0], pad]).reshape(TCHUNK, CHUNK)
    dstp = jnp.concatenate([edge_index[1], pad]).reshape(TCHUNK, CHUNK)
    zrows = jnp.zeros((RPT, HID), f32)
    zdeg = jnp.zeros((RPT, DEGW), f32)
    ones_deg = jnp.ones((CHUNK, DEGW), f32)

    degp = _deg_call(dstp, ones_deg, zdeg)
    h0, cpart = _tc1_call(xp, W1, Wc, bc.reshape(1, LAT))
    dinv, hs0 = _tc2_call(degp[:NP], degp[NP:], h0)
    accp = _agg_call(hs0, srcp, dstp, zrows)
    hs1 = _tc3_call(accp[:NP], accp[NP:], hs0, dinv, b1.reshape(1, HID))
    qp = _agg_call(hs1, srcp, dstp, zrows)
    mu, lv, hc, xpred, y = _tc4_call(
        qp[:NP], qp[NP:], hs1, dinv, cpart,
        Wmu, bmu.reshape(1, LAT), Wlv, blv.reshape(1, LAT),
        Wf, bf.reshape(1, FEAT), Wl1, bl1.reshape(1, HID),
        Wl2, bl2.reshape(1, NC))
    A = _tc5_call(hc[:N])
    mu = mu[:N]
    return (mu, lv[:N], mu, A, xpred[:N], y[:N])
